# baseline (speedup 1.0000x reference)
import functools

import jax
import jax.numpy as jnp
from jax import lax
from jax.experimental import pallas as pl
from jax.experimental.pallas import tpu as pltpu
from jax.experimental.pallas import tpu_sc as plsc

_B = 2
_NUM = 256
_CF = 128
_H = 4
_K = 32
_G = 4
_GW = _CF // _G
_ROWS = _B * _NUM
_NW = 32
_RPW = _ROWS // _NW
_L = 16


def _sc_mask_body(s_hbm, out_hbm, s2, st, cm):
    wid = lax.axis_index("s") * 2 + lax.axis_index("c")
    iota = lax.broadcasted_iota(jnp.int32, (_L,), 0)
    pltpu.sync_copy(s_hbm.at[pl.ds(wid * _RPW, _RPW), :], s2)
    for r in range(_RPW):
        for c in range(_NUM // _L):
            v = s2[r, pl.ds(c * _L, _L)]
            plsc.store_scatter(st, [iota * _L + (c * _NUM + r)], v)
    zeros = jnp.zeros((_L,), jnp.float32)
    for c in range(_NUM):
        cm[pl.ds(c * _L, _L)] = zeros

    ones = jnp.ones((_L,), jnp.float32)
    negones = jnp.full((_L,), -1.0, jnp.float32)

    def body(_, carry):
        _U = 8
        ms = [jnp.full((_L,), -1.0, jnp.float32) for _ in range(_U)]
        fs = [jnp.zeros((_L,), jnp.int32) for _ in range(_U)]
        for jg in range(_NUM // _U):
            for u in range(_U):
                j = jg * _U + u
                v = st[pl.ds(j * _L, _L)]
                gt = v > ms[u]
                ms[u] = jnp.maximum(ms[u], v)
                fs[u] = jnp.where(gt, j, fs[u])
        m = ms[0]
        for u in range(1, _U):
            m = jnp.maximum(m, ms[u])
        fidx = jnp.full((_L,), 2 * _NUM, jnp.int32)
        for u in range(_U):
            fidx = jnp.minimum(fidx, jnp.where(ms[u] == m, fs[u], 2 * _NUM))
        plsc.store_scatter(st, [fidx * _L + iota], negones)
        plsc.store_scatter(cm, [fidx + iota * _NUM], ones)
        return carry

    lax.fori_loop(0, _K, body, 0)
    pltpu.sync_copy(cm, out_hbm.at[wid])


def _sc_mask(S):
    mesh = plsc.VectorSubcoreMesh(core_axis_name="c", subcore_axis_name="s")
    fn = functools.partial(
        pl.kernel,
        out_type=jax.ShapeDtypeStruct((_NW, _RPW * _NUM), jnp.float32),
        mesh=mesh,
        compiler_params=pltpu.CompilerParams(needs_layout_passes=False),
        scratch_types=[
            pltpu.VMEM((_RPW, _NUM), jnp.float32),
            pltpu.VMEM((_RPW * _NUM,), jnp.float32),
            pltpu.VMEM((_RPW * _NUM,), jnp.float32),
        ],
    )(_sc_mask_body)
    return fn(S).reshape(_ROWS, _NUM)


def _gram(x):
    xT = jnp.transpose(x, (1, 0))
    inv_col = 1.0 / jnp.maximum(
        jnp.sqrt(jnp.sum(x * x, axis=1, keepdims=True)), 1e-8)
    inv_row = 1.0 / jnp.maximum(
        jnp.sqrt(jnp.sum(xT * xT, axis=0, keepdims=True)), 1e-8)
    G = jnp.dot(x, xT, preferred_element_type=jnp.float32)
    return jnp.maximum(G * inv_col * inv_row, 0.0), xT


def _gram_body(x_ref, s_ref):
    for b in range(_B):
        S, _ = _gram(x_ref[b])
        s_ref[pl.ds(b * _NUM, _NUM), :] = S


def _finish(x, xT, boxes, boxesT, roi, fcol, colmask, waT, was, wcx, wcs,
            brow, Wbd, cb, eye):
    aT = (jnp.dot(waT, xT, preferred_element_type=jnp.float32)
          + jnp.dot(was, boxesT, preferred_element_type=jnp.float32))
    cC = (jnp.dot(x, wcx, preferred_element_type=jnp.float32)
          + jnp.dot(boxes, wcs, preferred_element_type=jnp.float32) + brow)
    conv = jnp.maximum(
        jnp.dot(x, Wbd, preferred_element_type=jnp.float32) + cb, 0.0)
    pieces = []
    for h in range(_H):
        L = cC[:, h:h + 1] + aT[h:h + 1, :]
        P = jax.nn.sigmoid(L)
        M = (P * roi * colmask + fcol * eye) * 0.25
        pieces.append(jnp.dot(M, conv[:, h * _GW:(h + 1) * _GW],
                              preferred_element_type=jnp.float32))
    return conv + jnp.concatenate(pieces, axis=1)


def _mid_body(x_ref, boxes_ref, boxesT_ref, roi_ref, smrow_ref, smcol_ref,
              cm_ref, waT_ref, was_ref, wcx_ref, wcs_ref, b_ref, Wbd_ref,
              cb_ref, mid_ref, s2_ref):
    iota_j = lax.broadcasted_iota(jnp.int32, (_NUM, _NUM), 1)
    iota_i = lax.broadcasted_iota(jnp.int32, (_NUM, _NUM), 0)
    eye = jnp.where(iota_i == iota_j, 1.0, 0.0)
    colmask = jnp.max(cm_ref[...], axis=0, keepdims=True)
    for b in range(_B):
        x = x_ref[b]
        xT = jnp.transpose(x, (1, 0))
        roi = roi_ref[b] * smrow_ref[b]
        fcol = jnp.where(smcol_ref[b] == 0.0, 1.0, 0.0)
        mid = _finish(x, xT, boxes_ref[b], boxesT_ref[b], roi, fcol, colmask,
                      waT_ref[...], was_ref[...], wcx_ref[...], wcs_ref[...],
                      b_ref[...], Wbd_ref[...], cb_ref[...], eye)
        mid_ref[b] = mid
        S2, _ = _gram(mid)
        s2_ref[pl.ds(b * _NUM, _NUM), :] = S2


def _final_body(mid_ref, boxes_ref, boxesT_ref, roi_ref, smrow_ref, smcol_ref,
                cm_ref, waT_ref, was_ref, wcx_ref, wcs_ref, b_ref, Wbd_ref,
                cb_ref, lnw_ref, lnb_ref, out_ref):
    iota_j = lax.broadcasted_iota(jnp.int32, (_NUM, _NUM), 1)
    iota_i = lax.broadcasted_iota(jnp.int32, (_NUM, _NUM), 0)
    eye = jnp.where(iota_i == iota_j, 1.0, 0.0)
    colmask = jnp.max(cm_ref[...], axis=0, keepdims=True)
    lnw = lnw_ref[...]
    lnb = lnb_ref[...]
    for b in range(_B):
        x = mid_ref[b]
        xT = jnp.transpose(x, (1, 0))
        roi = roi_ref[b] * smrow_ref[b]
        fcol = jnp.where(smcol_ref[b] == 0.0, 1.0, 0.0)
        v = _finish(x, xT, boxes_ref[b], boxesT_ref[b], roi, fcol, colmask,
                    waT_ref[...], was_ref[...], wcx_ref[...], wcs_ref[...],
                    b_ref[...], Wbd_ref[...], cb_ref[...], eye)
        mu = jnp.mean(v, axis=1, keepdims=True)
        d = v - mu
        var = jnp.mean(d * d, axis=1, keepdims=True)
        out_ref[b] = d * lax.rsqrt(var + 1e-6) * lnw + lnb


def _blockdiag(w):
    z = jnp.zeros((_CF, _CF), jnp.float32)
    for g in range(_G):
        z = z.at[g * _GW:(g + 1) * _GW, g * _GW:(g + 1) * _GW].set(
            jnp.transpose(w[g * _GW:(g + 1) * _GW, :]))
    return z


def kernel(input, boxes, masks_roi, score_mask, lin1_w, lin1_b, lin2_w, lin2_b,
           conv1_w, conv1_b, conv2_w, conv2_b, ln_w, ln_b):
    x = input.astype(jnp.float32)
    boxesT = jnp.swapaxes(boxes, 1, 2)
    smrow = score_mask[:, None, :]
    smcol = score_mask[:, :, None]

    def split_lin(w):
        waT = w[:, :_CF]
        was = w[:, 2 * _CF:2 * _CF + 2]
        wcx = jnp.transpose(w[:, _CF:2 * _CF])
        wcs = jnp.transpose(w[:, 2 * _CF + 2:2 * _CF + 4])
        return waT, was, wcx, wcs

    wa1T, wa1s, wc1x, wc1s = split_lin(lin1_w)
    wa2T, wa2s, wc2x, wc2s = split_lin(lin2_w)
    b1 = lin1_b[None, :]
    b2 = lin2_b[None, :]
    Wbd1 = _blockdiag(conv1_w)
    Wbd2 = _blockdiag(conv2_w)
    cb1 = conv1_b[None, :]
    cb2 = conv2_b[None, :]
    lnw = ln_w[None, :]
    lnb = ln_b[None, :]

    S1 = pl.pallas_call(
        _gram_body,
        out_shape=jax.ShapeDtypeStruct((_ROWS, _NUM), jnp.float32),
    )(x)
    cm1 = _sc_mask(S1)

    mid, S2 = pl.pallas_call(
        _mid_body,
        out_shape=[jax.ShapeDtypeStruct((_B, _NUM, _CF), jnp.float32),
                   jax.ShapeDtypeStruct((_ROWS, _NUM), jnp.float32)],
    )(x, boxes, boxesT, masks_roi, smrow, smcol, cm1,
      wa1T, wa1s, wc1x, wc1s, b1, Wbd1, cb1)
    cm2 = _sc_mask(S2)

    return pl.pallas_call(
        _final_body,
        out_shape=jax.ShapeDtypeStruct((_B, _NUM, _CF), jnp.float32),
    )(mid, boxes, boxesT, masks_roi, smrow, smcol, cm2,
      wa2T, wa2s, wc2x, wc2s, b2, Wbd2, cb2, lnw, lnb)


# speedup vs baseline: 1.0930x; 1.0930x over previous
import functools

import jax
import jax.numpy as jnp
from jax import lax
from jax.experimental import pallas as pl
from jax.experimental.pallas import tpu as pltpu
from jax.experimental.pallas import tpu_sc as plsc

_B = 2
_NUM = 256
_CF = 128
_H = 4
_K = 32
_G = 4
_GW = _CF // _G
_ROWS = _B * _NUM
_NW = 32
_RPW = _ROWS // _NW
_L = 16


def _sc_mask_body(s_hbm, out_hbm, s2, st, sm, cm):
    wid = lax.axis_index("s") * 2 + lax.axis_index("c")
    iota = lax.broadcasted_iota(jnp.int32, (_L,), 0)
    pltpu.sync_copy(s_hbm.at[pl.ds(wid * _RPW, _RPW), :], s2)
    _NG = _NUM // _L
    for r in range(_RPW):
        rcol = jnp.full((_L,), r, jnp.int32)
        for c in range(_NG):
            v = s2[r, pl.ds(c * _L, _L)]
            plsc.store_scatter(st, [iota + c * _L, rcol], v)
    for g in range(_NG):
        gm = st[g * _L]
        for t in range(1, _L):
            gm = jnp.maximum(gm, st[g * _L + t])
        sm[g] = gm
    zeros = jnp.zeros((_L,), jnp.float32)
    for c in range(_NUM):
        cm[pl.ds(c * _L, _L)] = zeros

    ones = jnp.ones((_L,), jnp.float32)
    negones = jnp.full((_L,), -1.0, jnp.float32)

    def body(_, carry):
        m = sm[0]
        for g in range(1, _NG):
            m = jnp.maximum(m, sm[g])
        gw = jnp.full((_L,), _NG, jnp.int32)
        for g in range(_NG - 1, -1, -1):
            gw = jnp.where(sm[g] == m, g, gw)
        vals = []
        m2 = jnp.full((_L,), -1.0, jnp.float32)
        t2 = jnp.zeros((_L,), jnp.int32)
        for t in range(_L):
            v = plsc.load_gather(st, [gw * _L + t, iota])
            vals.append(v)
            gt = v > m2
            m2 = jnp.maximum(m2, v)
            t2 = jnp.where(gt, t, t2)
        fidx = gw * _L + t2
        plsc.store_scatter(st, [fidx, iota], negones)
        plsc.store_scatter(cm, [fidx + iota * _NUM], ones)
        ngm = negones
        for t in range(_L):
            ngm = jnp.maximum(ngm, jnp.where(t2 == t, negones, vals[t]))
        plsc.store_scatter(sm, [gw, iota], ngm)
        return carry

    lax.fori_loop(0, _K, body, 0)
    pltpu.sync_copy(cm, out_hbm.at[wid])


def _sc_mask(S):
    mesh = plsc.VectorSubcoreMesh(core_axis_name="c", subcore_axis_name="s")
    fn = functools.partial(
        pl.kernel,
        out_type=jax.ShapeDtypeStruct((_NW, _RPW * _NUM), jnp.float32),
        mesh=mesh,
        compiler_params=pltpu.CompilerParams(needs_layout_passes=False),
        scratch_types=[
            pltpu.VMEM((_RPW, _NUM), jnp.float32),
            pltpu.VMEM((_NUM, _L), jnp.float32),
            pltpu.VMEM((_NUM // _L, _L), jnp.float32),
            pltpu.VMEM((_RPW * _NUM,), jnp.float32),
        ],
    )(_sc_mask_body)
    return fn(S).reshape(_ROWS, _NUM)


def _gram(x):
    xT = jnp.transpose(x, (1, 0))
    inv_col = 1.0 / jnp.maximum(
        jnp.sqrt(jnp.sum(x * x, axis=1, keepdims=True)), 1e-8)
    inv_row = 1.0 / jnp.maximum(
        jnp.sqrt(jnp.sum(xT * xT, axis=0, keepdims=True)), 1e-8)
    G = jnp.dot(x, xT, preferred_element_type=jnp.float32)
    return jnp.maximum(G * inv_col * inv_row, 0.0), xT


def _gram_body(x_ref, s_ref):
    for b in range(_B):
        S, _ = _gram(x_ref[b])
        s_ref[pl.ds(b * _NUM, _NUM), :] = S


def _finish(x, xT, boxes, boxesT, roi, fcol, colmask, waT, was, wcx, wcs,
            brow, Wbd, cb, eye):
    aT = (jnp.dot(waT, xT, preferred_element_type=jnp.float32)
          + jnp.dot(was, boxesT, preferred_element_type=jnp.float32))
    cC = (jnp.dot(x, wcx, preferred_element_type=jnp.float32)
          + jnp.dot(boxes, wcs, preferred_element_type=jnp.float32) + brow)
    conv = jnp.maximum(
        jnp.dot(x, Wbd, preferred_element_type=jnp.float32) + cb, 0.0)
    pieces = []
    for h in range(_H):
        L = cC[:, h:h + 1] + aT[h:h + 1, :]
        P = jax.nn.sigmoid(L)
        M = (P * roi * colmask + fcol * eye) * 0.25
        pieces.append(jnp.dot(M, conv[:, h * _GW:(h + 1) * _GW],
                              preferred_element_type=jnp.float32))
    return conv + jnp.concatenate(pieces, axis=1)


def _mid_body(x_ref, boxes_ref, boxesT_ref, roi_ref, smrow_ref, smcol_ref,
              cm_ref, waT_ref, was_ref, wcx_ref, wcs_ref, b_ref, Wbd_ref,
              cb_ref, mid_ref, s2_ref):
    iota_j = lax.broadcasted_iota(jnp.int32, (_NUM, _NUM), 1)
    iota_i = lax.broadcasted_iota(jnp.int32, (_NUM, _NUM), 0)
    eye = jnp.where(iota_i == iota_j, 1.0, 0.0)
    colmask = jnp.max(cm_ref[...], axis=0, keepdims=True)
    for b in range(_B):
        x = x_ref[b]
        xT = jnp.transpose(x, (1, 0))
        roi = roi_ref[b] * smrow_ref[b]
        fcol = jnp.where(smcol_ref[b] == 0.0, 1.0, 0.0)
        mid = _finish(x, xT, boxes_ref[b], boxesT_ref[b], roi, fcol, colmask,
                      waT_ref[...], was_ref[...], wcx_ref[...], wcs_ref[...],
                      b_ref[...], Wbd_ref[...], cb_ref[...], eye)
        mid_ref[b] = mid
        S2, _ = _gram(mid)
        s2_ref[pl.ds(b * _NUM, _NUM), :] = S2


def _final_body(mid_ref, boxes_ref, boxesT_ref, roi_ref, smrow_ref, smcol_ref,
                cm_ref, waT_ref, was_ref, wcx_ref, wcs_ref, b_ref, Wbd_ref,
                cb_ref, lnw_ref, lnb_ref, out_ref):
    iota_j = lax.broadcasted_iota(jnp.int32, (_NUM, _NUM), 1)
    iota_i = lax.broadcasted_iota(jnp.int32, (_NUM, _NUM), 0)
    eye = jnp.where(iota_i == iota_j, 1.0, 0.0)
    colmask = jnp.max(cm_ref[...], axis=0, keepdims=True)
    lnw = lnw_ref[...]
    lnb = lnb_ref[...]
    for b in range(_B):
        x = mid_ref[b]
        xT = jnp.transpose(x, (1, 0))
        roi = roi_ref[b] * smrow_ref[b]
        fcol = jnp.where(smcol_ref[b] == 0.0, 1.0, 0.0)
        v = _finish(x, xT, boxes_ref[b], boxesT_ref[b], roi, fcol, colmask,
                    waT_ref[...], was_ref[...], wcx_ref[...], wcs_ref[...],
                    b_ref[...], Wbd_ref[...], cb_ref[...], eye)
        mu = jnp.mean(v, axis=1, keepdims=True)
        d = v - mu
        var = jnp.mean(d * d, axis=1, keepdims=True)
        out_ref[b] = d * lax.rsqrt(var + 1e-6) * lnw + lnb


def _blockdiag(w):
    z = jnp.zeros((_CF, _CF), jnp.float32)
    for g in range(_G):
        z = z.at[g * _GW:(g + 1) * _GW, g * _GW:(g + 1) * _GW].set(
            jnp.transpose(w[g * _GW:(g + 1) * _GW, :]))
    return z


def kernel(input, boxes, masks_roi, score_mask, lin1_w, lin1_b, lin2_w, lin2_b,
           conv1_w, conv1_b, conv2_w, conv2_b, ln_w, ln_b):
    x = input.astype(jnp.float32)
    boxesT = jnp.swapaxes(boxes, 1, 2)
    smrow = score_mask[:, None, :]
    smcol = score_mask[:, :, None]

    def split_lin(w):
        waT = w[:, :_CF]
        was = w[:, 2 * _CF:2 * _CF + 2]
        wcx = jnp.transpose(w[:, _CF:2 * _CF])
        wcs = jnp.transpose(w[:, 2 * _CF + 2:2 * _CF + 4])
        return waT, was, wcx, wcs

    wa1T, wa1s, wc1x, wc1s = split_lin(lin1_w)
    wa2T, wa2s, wc2x, wc2s = split_lin(lin2_w)
    b1 = lin1_b[None, :]
    b2 = lin2_b[None, :]
    Wbd1 = _blockdiag(conv1_w)
    Wbd2 = _blockdiag(conv2_w)
    cb1 = conv1_b[None, :]
    cb2 = conv2_b[None, :]
    lnw = ln_w[None, :]
    lnb = ln_b[None, :]

    S1 = pl.pallas_call(
        _gram_body,
        out_shape=jax.ShapeDtypeStruct((_ROWS, _NUM), jnp.float32),
    )(x)
    cm1 = _sc_mask(S1)

    mid, S2 = pl.pallas_call(
        _mid_body,
        out_shape=[jax.ShapeDtypeStruct((_B, _NUM, _CF), jnp.float32),
                   jax.ShapeDtypeStruct((_ROWS, _NUM), jnp.float32)],
    )(x, boxes, boxesT, masks_roi, smrow, smcol, cm1,
      wa1T, wa1s, wc1x, wc1s, b1, Wbd1, cb1)
    cm2 = _sc_mask(S2)

    return pl.pallas_call(
        _final_body,
        out_shape=jax.ShapeDtypeStruct((_B, _NUM, _CF), jnp.float32),
    )(mid, boxes, boxesT, masks_roi, smrow, smcol, cm2,
      wa2T, wa2s, wc2x, wc2s, b2, Wbd2, cb2, lnw, lnb)
